# baseline (speedup 1.0000x reference)
import jax
import jax.numpy as jnp
from jax.experimental import pallas as pl

MASK_PROB = 0.15
MASK_TOKEN = 103

BLOCK_ROWS = 512


def _mask_kernel(ids_ref, rand_ref, masked_ref, labels_ref):
    ids = ids_ref[...]
    rand = rand_ref[...]
    mask = rand < MASK_PROB
    masked = jnp.where(mask, jnp.int32(MASK_TOKEN), ids)
    masked_ref[...] = masked
    labels_ref[...] = jnp.where(masked == MASK_TOKEN, ids, jnp.int32(-100))


def kernel(input_ids, rand_vals):
    n_rows, n_cols = input_ids.shape
    grid = (n_rows // BLOCK_ROWS, 2)
    in_spec = pl.BlockSpec((BLOCK_ROWS, n_cols // 2), lambda i, j: (i, j))
    out_spec = pl.BlockSpec((BLOCK_ROWS, n_cols // 2), lambda i, j: (i, j))
    out_shape = jax.ShapeDtypeStruct(input_ids.shape, input_ids.dtype)
    masked, labels = pl.pallas_call(
        _mask_kernel,
        grid=grid,
        in_specs=[in_spec, in_spec],
        out_specs=[out_spec, out_spec],
        out_shape=[out_shape, out_shape],
    )(input_ids, rand_vals)
    return masked, labels


# speedup vs baseline: 1.0357x; 1.0357x over previous
import jax
import jax.numpy as jnp
from jax.experimental import pallas as pl

MASK_PROB = 0.15
MASK_TOKEN = 103

BLOCK_ROWS = 512


def _mask_kernel(ids_ref, rand_ref, masked_ref, labels_ref):
    ids = ids_ref[...]
    rand = rand_ref[...]
    mask = rand < MASK_PROB
    masked = jnp.where(mask, jnp.int32(MASK_TOKEN), ids)
    masked_ref[...] = masked
    labels_ref[...] = jnp.where(masked == MASK_TOKEN, ids, jnp.int32(-100))


def kernel(input_ids, rand_vals):
    n_rows, n_cols = input_ids.shape
    grid = (n_rows // BLOCK_ROWS,)
    spec = pl.BlockSpec((BLOCK_ROWS, n_cols), lambda i: (i, 0))
    out_shape = jax.ShapeDtypeStruct(input_ids.shape, input_ids.dtype)
    masked, labels = pl.pallas_call(
        _mask_kernel,
        grid=grid,
        in_specs=[spec, spec],
        out_specs=[spec, spec],
        out_shape=[out_shape, out_shape],
    )(input_ids, rand_vals)
    return masked, labels
